# baseline (speedup 1.0000x reference)
import functools

import jax
import jax.numpy as jnp
from jax import lax
from jax.experimental import pallas as pl
from jax.experimental.pallas import tpu as pltpu
from jax.experimental.pallas import tpu_sc as plsc

NC = 2
NS = 16
NW = NC * NS
LANE = 16
CHUNK = 2048
CROWS = 16
CCOLS = CHUNK // CROWS


def _mesh():
    return plsc.VectorSubcoreMesh(core_axis_name="c", subcore_axis_name="s")


def _cparams():
    return pltpu.CompilerParams(needs_layout_passes=False)


def _worker_id():
    return lax.axis_index("c") * NS + lax.axis_index("s")


def _f32(shape):
    return jax.ShapeDtypeStruct(shape, jnp.float32)


def _zero_fill(ref, n):
    def body(i, _):
        ref[pl.ds(i * LANE, LANE)] = jnp.zeros((LANE,), jnp.float32)
        return 0
    lax.fori_loop(0, n // LANE, body, 0)


def _make_extract(npad, n):
    per = npad // NW
    groups = per // LANE
    tail = n - per * (NW - 1)

    def body(xf_hbm, af_hbm, xs_hbm, a0_hbm, inbuf, outbuf):
        w = _worker_id()
        iota = lax.iota(jnp.int32, LANE)
        for src_hbm, dst_hbm in ((xf_hbm, xs_hbm), (af_hbm, a0_hbm)):
            @pl.when(w < NW - 1)
            def _():
                pltpu.sync_copy(src_hbm.at[pl.ds(w * per * 4, per * 4)],
                                inbuf)

            @pl.when(w == NW - 1)
            def _():
                pltpu.sync_copy(
                    src_hbm.at[pl.ds((NW - 1) * per * 4, tail * 4)],
                    inbuf.at[pl.ds(0, tail * 4)])

            def grp(g, _):
                idx = g * (4 * LANE) + 4 * iota
                outbuf[pl.ds(g * LANE, LANE)] = plsc.load_gather(inbuf, [idx])
                return 0
            lax.fori_loop(0, groups, grp, 0)
            pltpu.sync_copy(outbuf, dst_hbm.at[pl.ds(w * per, per)])

    return pl.kernel(
        body,
        out_type=(_f32((npad,)), _f32((npad,))),
        mesh=_mesh(), compiler_params=_cparams(),
        scratch_types=[
            pltpu.VMEM((per * 4,), jnp.float32),
            pltpu.VMEM((per,), jnp.float32),
        ],
    )


def _make_edge_pass(npad, n_edges, with_count):
    nchunk = n_edges // CHUNK
    ch_per_w = -(-nchunk // NW)
    sper = npad // NS

    def body(x_hbm, ei_hbm, ea_hbm, *rest):
        if with_count:
            (pa_hbm, pc_hbm, acc_sh, cnt_sh, x_local, src2d, dst2d,
             attr, vals, ones, sem, sem2) = rest
        else:
            (pa_hbm, acc_sh, x_local, src2d, dst2d,
             attr, vals, sem) = rest
        iobuf = attr.at[pl.ds(0, sper)]
        c = lax.axis_index("c")
        s = lax.axis_index("s")
        w = c * NS + s
        iota = lax.iota(jnp.int32, LANE)

        _zero_fill(attr, sper)
        pltpu.sync_copy(iobuf, acc_sh.at[pl.ds(s * sper, sper)])
        if with_count:
            pltpu.sync_copy(iobuf, cnt_sh.at[pl.ds(s * sper, sper)])

            def ob(i, _):
                r = i // (CCOLS // LANE)
                col = (i % (CCOLS // LANE)) * LANE
                ones[r, pl.ds(col, LANE)] = jnp.ones((LANE,), jnp.float32)
                return 0
            lax.fori_loop(0, CHUNK // LANE, ob, 0)

        pltpu.sync_copy(x_hbm, x_local)
        plsc.subcore_barrier()

        def chunk_body(j, _):
            chunk = w + NW * j

            @pl.when(chunk < nchunk)
            def _():
                pltpu.sync_copy(ei_hbm.at[0, chunk], src2d)
                pltpu.sync_copy(ei_hbm.at[1, chunk], dst2d)
                pltpu.sync_copy(
                    ea_hbm.at[pl.ds(chunk * (4 * CHUNK), 4 * CHUNK)], attr)

                def grp(g, _):
                    r = g // (CCOLS // LANE)
                    col = (g % (CCOLS // LANE)) * LANE
                    si = src2d[r, pl.ds(col, LANE)]
                    di = dst2d[r, pl.ds(col, LANE)]
                    sv = plsc.load_gather(x_local, [si])
                    dv = plsc.load_gather(x_local, [di])
                    abase = g * (4 * LANE) + 4 * iota
                    e0 = plsc.load_gather(attr, [abase])
                    if with_count:
                        wgt = 1.0 / e0
                    else:
                        e1 = plsc.load_gather(attr, [abase + 1])
                        wgt = (e0 + e1) / (e0 * e1)
                    vals[r, pl.ds(col, LANE)] = (dv - sv) * wgt
                    return 0
                lax.fori_loop(0, CHUNK // LANE, grp, 0)

                handles = []
                for r in range(CROWS):
                    handles.append(pltpu.async_copy(
                        vals.at[r], acc_sh.at[dst2d.at[r]], sem, add=True))
                    if with_count:
                        handles.append(pltpu.async_copy(
                            ones.at[r], cnt_sh.at[dst2d.at[r]], sem2,
                            add=True))
                for h in handles:
                    h.wait()
            return 0
        lax.fori_loop(0, ch_per_w, chunk_body, 0)

        plsc.subcore_barrier()
        pltpu.sync_copy(acc_sh.at[pl.ds(s * sper, sper)], iobuf)
        pltpu.sync_copy(iobuf, pa_hbm.at[pl.ds(c * npad + s * sper, sper)])
        if with_count:
            pltpu.sync_copy(cnt_sh.at[pl.ds(s * sper, sper)], iobuf)
            pltpu.sync_copy(iobuf, pc_hbm.at[pl.ds(c * npad + s * sper, sper)])

    out_type = (_f32((NC * npad,)), _f32((NC * npad,))) if with_count \
        else _f32((NC * npad,))
    scratch = [pltpu.VMEM_SHARED((npad,), jnp.float32)]
    if with_count:
        scratch.append(pltpu.VMEM_SHARED((npad,), jnp.float32))
    scratch += [
        pltpu.VMEM((npad,), jnp.float32),
        pltpu.VMEM((CROWS, CCOLS), jnp.int32),
        pltpu.VMEM((CROWS, CCOLS), jnp.int32),
        pltpu.VMEM((4 * CHUNK,), jnp.float32),
        pltpu.VMEM((CROWS, CCOLS), jnp.float32),
    ]
    if with_count:
        scratch.append(pltpu.VMEM((CROWS, CCOLS), jnp.float32))
    scratch.append(pltpu.SemaphoreType.DMA)
    if with_count:
        scratch.append(pltpu.SemaphoreType.DMA)

    return pl.kernel(
        body, out_type=out_type, mesh=_mesh(), compiler_params=_cparams(),
        scratch_types=scratch)


def _make_glue_t(npad):
    per = npad // NW
    groups = per // LANE

    def body(pa_hbm, pc_hbm, a0_hbm, t_hbm, cnt_hbm,
             b0, b1, c0, c1, ab, tb, cb):
        w = _worker_id()
        sl = pl.ds(w * per, per)
        sl1 = pl.ds(npad + w * per, per)
        pltpu.sync_copy(pa_hbm.at[sl], b0)
        pltpu.sync_copy(pa_hbm.at[sl1], b1)
        pltpu.sync_copy(pc_hbm.at[sl], c0)
        pltpu.sync_copy(pc_hbm.at[sl1], c1)
        pltpu.sync_copy(a0_hbm.at[sl], ab)

        def grp(g, _):
            d = pl.ds(g * LANE, LANE)
            cc = c0[d] + c1[d]
            t = ab[d] * (b0[d] + b1[d]) / jnp.maximum(cc, 1.0)
            tb[d] = t
            cb[d] = cc
            return 0
        lax.fori_loop(0, groups, grp, 0)
        pltpu.sync_copy(tb, t_hbm.at[sl])
        pltpu.sync_copy(cb, cnt_hbm.at[sl])

    return pl.kernel(
        body,
        out_type=(_f32((npad,)), _f32((npad,))),
        mesh=_mesh(), compiler_params=_cparams(),
        scratch_types=[pltpu.VMEM((per,), jnp.float32) for _ in range(7)],
    )


def _make_final(npad, n):
    per = npad // NW
    groups = per // LANE
    tail = n - per * (NW - 1)

    def body(pb_hbm, cnt_hbm, mask_hbm, fm_hbm, o_hbm,
             b0, b1, cb, mb, fb, ob):
        w = _worker_id()
        sl = pl.ds(w * per, per)
        pltpu.sync_copy(pb_hbm.at[sl], b0)
        pltpu.sync_copy(pb_hbm.at[pl.ds(npad + w * per, per)], b1)
        pltpu.sync_copy(cnt_hbm.at[sl], cb)

        @pl.when(w < NW - 1)
        def _():
            pltpu.sync_copy(mask_hbm.at[sl], mb)
            pltpu.sync_copy(fm_hbm.at[sl], fb)

        @pl.when(w == NW - 1)
        def _():
            tsl = pl.ds((NW - 1) * per, tail)
            pltpu.sync_copy(mask_hbm.at[tsl], mb.at[pl.ds(0, tail)])
            pltpu.sync_copy(fm_hbm.at[tsl], fb.at[pl.ds(0, tail)])

        def grp(g, _):
            d = pl.ds(g * LANE, LANE)
            v = (b0[d] + b1[d]) / jnp.maximum(cb[d], 1.0)
            ob[d] = v * mb[d] - fb[d]
            return 0
        lax.fori_loop(0, groups, grp, 0)
        pltpu.sync_copy(ob, o_hbm.at[sl])

    return pl.kernel(
        body,
        out_type=_f32((npad,)),
        mesh=_mesh(), compiler_params=_cparams(),
        scratch_types=[pltpu.VMEM((per,), jnp.float32) for _ in range(6)],
    )


def _pde_loss(out_x, a_x_x, edge_index, edge_attr, mask, f):
    n = out_x.shape[0]
    n_edges = edge_index.shape[1]
    npad = -(-n // (NW * LANE)) * (NW * LANE)

    xf = out_x.reshape(-1)
    af = a_x_x.reshape(-1)
    fm = mask * jnp.asarray(f, jnp.float32)
    ei4 = edge_index.reshape(2, n_edges // CHUNK, CROWS, CCOLS)
    ea_f = edge_attr.reshape(-1)

    xs, a0 = _make_extract(npad, n)(xf, af)
    pa, pc = _make_edge_pass(npad, n_edges, True)(xs, ei4, ea_f)
    t, cnt = _make_glue_t(npad)(pa, pc, a0)
    pb = _make_edge_pass(npad, n_edges, False)(t, ei4, ea_f)
    out = _make_final(npad, n)(pb, cnt, mask, fm)
    return out[:n]


def kernel(out_x, a_x_x, edge_index, edge_attr, mask, f):
    return _pde_loss(out_x, a_x_x, edge_index, edge_attr, mask, f)


# speedup vs baseline: 9.6543x; 9.6543x over previous
import jax
import jax.numpy as jnp
from jax import lax
from jax.experimental import pallas as pl
from jax.experimental.pallas import tpu as pltpu
from jax.experimental.pallas import tpu_sc as plsc

NC = 2
NS = 16
NW = NC * NS
LANE = 16
CHUNK = 2048
CROWS = 16
CCOLS = CHUNK // CROWS
GRPS_PER_ROW = CCOLS // LANE


def _mesh():
    return plsc.VectorSubcoreMesh(core_axis_name="c", subcore_axis_name="s")


def _cparams():
    return pltpu.CompilerParams(needs_layout_passes=False)


def _worker_id():
    return lax.axis_index("c") * NS + lax.axis_index("s")


def _f32(shape):
    return jax.ShapeDtypeStruct(shape, jnp.float32)


def _zero_fill(ref, n):
    def body(i, _):
        ref[pl.ds(i * LANE, LANE)] = jnp.zeros((LANE,), jnp.float32)
        return 0
    lax.fori_loop(0, n // LANE, body, 0)


def _make_edge_pass(n, npad, n_edges, with_count):
    nchunk = n_edges // CHUNK
    ch_per_w = -(-nchunk // NW)
    sper = npad // NS

    def body(x_hbm, ei_hbm, *rest):
        if with_count:
            (e0_hbm, pa_hbm, pc_hbm, acc_sh, cnt_sh, x_local, src2d, dst2d,
             e0b, vals, ones, sem, sem2) = rest
        else:
            (e0_hbm, e1_hbm, pa_hbm, acc_sh, x_local, src2d, dst2d,
             e0b, e1b, vals, sem) = rest
        c = lax.axis_index("c")
        s = lax.axis_index("s")
        w = c * NS + s

        _zero_fill(x_local, sper)
        pltpu.sync_copy(x_local.at[pl.ds(0, sper)],
                        acc_sh.at[pl.ds(s * sper, sper)])
        if with_count:
            pltpu.sync_copy(x_local.at[pl.ds(0, sper)],
                            cnt_sh.at[pl.ds(s * sper, sper)])

            def ob(i, _):
                r = i // GRPS_PER_ROW
                col = (i % GRPS_PER_ROW) * LANE
                ones[r, pl.ds(col, LANE)] = jnp.ones((LANE,), jnp.float32)
                return 0
            lax.fori_loop(0, CHUNK // LANE, ob, 0)

        pltpu.sync_copy(x_hbm.at[pl.ds(0, n)], x_local.at[pl.ds(0, n)])
        plsc.subcore_barrier()

        def chunk_body(j, _):
            chunk = w + NW * j

            @pl.when(chunk < nchunk)
            def _():
                esl = pl.ds(chunk * CHUNK, CHUNK)
                pltpu.sync_copy(ei_hbm.at[0, chunk], src2d)
                pltpu.sync_copy(ei_hbm.at[1, chunk], dst2d)
                pltpu.sync_copy(e0_hbm.at[esl], e0b)
                if not with_count:
                    pltpu.sync_copy(e1_hbm.at[esl], e1b)

                def grp(g, _):
                    r = g // GRPS_PER_ROW
                    col = (g % GRPS_PER_ROW) * LANE
                    si = src2d[r, pl.ds(col, LANE)]
                    di = dst2d[r, pl.ds(col, LANE)]
                    sv = plsc.load_gather(x_local, [si])
                    dv = plsc.load_gather(x_local, [di])
                    e0 = e0b[pl.ds(g * LANE, LANE)]
                    if with_count:
                        wgt = 1.0 / e0
                    else:
                        e1 = e1b[pl.ds(g * LANE, LANE)]
                        wgt = (e0 + e1) / (e0 * e1)
                    vals[r, pl.ds(col, LANE)] = (dv - sv) * wgt
                    return 0
                lax.fori_loop(0, CHUNK // LANE, grp, 0)

                handles = []
                for r in range(CROWS):
                    handles.append(pltpu.async_copy(
                        vals.at[r], acc_sh.at[dst2d.at[r]], sem, add=True))
                    if with_count:
                        handles.append(pltpu.async_copy(
                            ones.at[r], cnt_sh.at[dst2d.at[r]], sem2,
                            add=True))
                for h in handles:
                    h.wait()
            return 0
        lax.fori_loop(0, ch_per_w, chunk_body, 0)

        plsc.subcore_barrier()
        pltpu.sync_copy(acc_sh.at[pl.ds(s * sper, sper)],
                        x_local.at[pl.ds(0, sper)])
        pltpu.sync_copy(x_local.at[pl.ds(0, sper)],
                        pa_hbm.at[pl.ds(c * npad + s * sper, sper)])
        if with_count:
            pltpu.sync_copy(cnt_sh.at[pl.ds(s * sper, sper)],
                            x_local.at[pl.ds(0, sper)])
            pltpu.sync_copy(x_local.at[pl.ds(0, sper)],
                            pc_hbm.at[pl.ds(c * npad + s * sper, sper)])

    out_type = (_f32((NC * npad,)), _f32((NC * npad,))) if with_count \
        else _f32((NC * npad,))
    scratch = [pltpu.VMEM_SHARED((npad,), jnp.float32)]
    if with_count:
        scratch.append(pltpu.VMEM_SHARED((npad,), jnp.float32))
    scratch += [
        pltpu.VMEM((npad,), jnp.float32),
        pltpu.VMEM((CROWS, CCOLS), jnp.int32),
        pltpu.VMEM((CROWS, CCOLS), jnp.int32),
        pltpu.VMEM((CHUNK,), jnp.float32),
    ]
    if not with_count:
        scratch.append(pltpu.VMEM((CHUNK,), jnp.float32))
    scratch.append(pltpu.VMEM((CROWS, CCOLS), jnp.float32))
    if with_count:
        scratch.append(pltpu.VMEM((CROWS, CCOLS), jnp.float32))
    scratch.append(pltpu.SemaphoreType.DMA)
    if with_count:
        scratch.append(pltpu.SemaphoreType.DMA)

    return pl.kernel(
        body, out_type=out_type, mesh=_mesh(), compiler_params=_cparams(),
        scratch_types=scratch)


def _make_glue_t(n, npad):
    per = npad // NW
    groups = per // LANE
    tail = n - per * (NW - 1)

    def body(pa_hbm, pc_hbm, a0_hbm, t_hbm, cnt_hbm,
             b0, b1, c0, c1, ab, tb, cb):
        w = _worker_id()
        sl = pl.ds(w * per, per)
        sl1 = pl.ds(npad + w * per, per)
        pltpu.sync_copy(pa_hbm.at[sl], b0)
        pltpu.sync_copy(pa_hbm.at[sl1], b1)
        pltpu.sync_copy(pc_hbm.at[sl], c0)
        pltpu.sync_copy(pc_hbm.at[sl1], c1)

        @pl.when(w < NW - 1)
        def _():
            pltpu.sync_copy(a0_hbm.at[sl], ab)

        @pl.when(w == NW - 1)
        def _():
            pltpu.sync_copy(a0_hbm.at[pl.ds((NW - 1) * per, tail)],
                            ab.at[pl.ds(0, tail)])

        def grp(g, _):
            d = pl.ds(g * LANE, LANE)
            cc = c0[d] + c1[d]
            t = ab[d] * (b0[d] + b1[d]) / jnp.maximum(cc, 1.0)
            tb[d] = t
            cb[d] = cc
            return 0
        lax.fori_loop(0, groups, grp, 0)
        pltpu.sync_copy(tb, t_hbm.at[sl])
        pltpu.sync_copy(cb, cnt_hbm.at[sl])

    return pl.kernel(
        body,
        out_type=(_f32((npad,)), _f32((npad,))),
        mesh=_mesh(), compiler_params=_cparams(),
        scratch_types=[pltpu.VMEM((per,), jnp.float32) for _ in range(7)],
    )


def _make_final(n, npad):
    per = npad // NW
    groups = per // LANE
    tail = n - per * (NW - 1)

    def body(pb_hbm, cnt_hbm, mask_hbm, fm_hbm, o_hbm,
             b0, b1, cb, mb, fb, ob):
        w = _worker_id()
        sl = pl.ds(w * per, per)
        pltpu.sync_copy(pb_hbm.at[sl], b0)
        pltpu.sync_copy(pb_hbm.at[pl.ds(npad + w * per, per)], b1)
        pltpu.sync_copy(cnt_hbm.at[sl], cb)

        @pl.when(w < NW - 1)
        def _():
            pltpu.sync_copy(mask_hbm.at[sl], mb)
            pltpu.sync_copy(fm_hbm.at[sl], fb)

        @pl.when(w == NW - 1)
        def _():
            tsl = pl.ds((NW - 1) * per, tail)
            pltpu.sync_copy(mask_hbm.at[tsl], mb.at[pl.ds(0, tail)])
            pltpu.sync_copy(fm_hbm.at[tsl], fb.at[pl.ds(0, tail)])

        def grp(g, _):
            d = pl.ds(g * LANE, LANE)
            v = (b0[d] + b1[d]) / jnp.maximum(cb[d], 1.0)
            ob[d] = v * mb[d] - fb[d]
            return 0
        lax.fori_loop(0, groups, grp, 0)
        pltpu.sync_copy(ob, o_hbm.at[sl])

    return pl.kernel(
        body,
        out_type=_f32((npad,)),
        mesh=_mesh(), compiler_params=_cparams(),
        scratch_types=[pltpu.VMEM((per,), jnp.float32) for _ in range(6)],
    )


def _pde_loss(out_x, a_x_x, edge_index, edge_attr, mask, f):
    n = out_x.shape[0]
    n_edges = edge_index.shape[1]
    npad = -(-n // (NW * LANE)) * (NW * LANE)

    x0 = out_x[:, 0]
    a0 = a_x_x[:, 0]
    e0 = edge_attr[:, 0]
    e1 = edge_attr[:, 1]
    fm = mask * jnp.asarray(f, jnp.float32)
    ei4 = edge_index.reshape(2, n_edges // CHUNK, CROWS, CCOLS)

    pa, pc = _make_edge_pass(n, npad, n_edges, True)(x0, ei4, e0)
    t, cnt = _make_glue_t(n, npad)(pa, pc, a0)
    pb = _make_edge_pass(n, npad, n_edges, False)(t, ei4, e0, e1)
    out = _make_final(n, npad)(pb, cnt, mask, fm)
    return out[:n]


def kernel(out_x, a_x_x, edge_index, edge_attr, mask, f):
    return _pde_loss(out_x, a_x_x, edge_index, edge_attr, mask, f)


# speedup vs baseline: 16.9646x; 1.7572x over previous
import jax
import jax.numpy as jnp
from jax import lax
from jax.experimental import pallas as pl
from jax.experimental.pallas import tpu as pltpu
from jax.experimental.pallas import tpu_sc as plsc

NC = 2
NS = 16
NW = NC * NS
LANE = 16
CHUNK = 2048
CROWS = 16
CCOLS = CHUNK // CROWS
GRPS_PER_ROW = CCOLS // LANE


def _mesh():
    return plsc.VectorSubcoreMesh(core_axis_name="c", subcore_axis_name="s")


def _cparams():
    return pltpu.CompilerParams(needs_layout_passes=False)


def _worker_id():
    return lax.axis_index("c") * NS + lax.axis_index("s")


def _f32(shape):
    return jax.ShapeDtypeStruct(shape, jnp.float32)


def _zero_fill(ref, n):
    def body(i, _):
        ref[pl.ds(i * LANE, LANE)] = jnp.zeros((LANE,), jnp.float32)
        return 0
    lax.fori_loop(0, n // LANE, body, 0)


def _make_edge_pass(n, npad, n_edges, with_count):
    nchunk = n_edges // CHUNK
    ch_per_w = -(-nchunk // NW)
    if ch_per_w % 2:
        ch_per_w += 1
    sper = npad // NS

    def body(x_hbm, ei_hbm, *rest):
        if with_count:
            (e0_hbm, pa_hbm, pc_hbm, acc_sh, cnt_sh, x_local,
             src0, src1, dst0, dst1, e00, e01, vals0, vals1, ones,
             semi0, semi1, semv0, semv1, semo0, semo1) = rest
        else:
            (e0_hbm, pa_hbm, acc_sh, x_local,
             src0, src1, dst0, dst1, e00, e01, vals0, vals1,
             semi0, semi1, semv0, semv1) = rest
            semo0 = semo1 = ones = None
        slots = (
            dict(src=src0, dst=dst0, e0=e00, vals=vals0,
                 semi=semi0, semv=semv0, semo=semo0),
            dict(src=src1, dst=dst1, e0=e01, vals=vals1,
                 semi=semi1, semv=semv1, semo=semo1),
        )
        c = lax.axis_index("c")
        s = lax.axis_index("s")
        w = c * NS + s

        _zero_fill(x_local, sper)
        pltpu.sync_copy(x_local.at[pl.ds(0, sper)],
                        acc_sh.at[pl.ds(s * sper, sper)])
        if with_count:
            pltpu.sync_copy(x_local.at[pl.ds(0, sper)],
                            cnt_sh.at[pl.ds(s * sper, sper)])

            def ob(i, _):
                r = i // GRPS_PER_ROW
                col = (i % GRPS_PER_ROW) * LANE
                ones[r, pl.ds(col, LANE)] = jnp.ones((LANE,), jnp.float32)
                return 0
            lax.fori_loop(0, CHUNK // LANE, ob, 0)

        def in_pairs(chunk, sl):
            esl = pl.ds(chunk * CHUNK, CHUNK)
            return [
                (ei_hbm.at[0, chunk], sl["src"]),
                (ei_hbm.at[1, chunk], sl["dst"]),
                (e0_hbm.at[esl], sl["e0"]),
            ]

        def start_in(jj, sl):
            chunk = w + NW * jj

            @pl.when(chunk < nchunk)
            def _():
                for a, b in in_pairs(chunk, sl):
                    pltpu.async_copy(a, b, sl["semi"])

        def wait_in(jj, sl):
            chunk = w + NW * jj

            @pl.when(chunk < nchunk)
            def _():
                for a, b in in_pairs(chunk, sl):
                    pltpu.make_async_copy(a, b, sl["semi"]).wait()

        def compute(jj, sl):
            chunk = w + NW * jj

            @pl.when(chunk < nchunk)
            def _():
                def row(r, _):
                    for kk in range(GRPS_PER_ROW):
                        cs = pl.ds(kk * LANE, LANE)
                        si = sl["src"][r, cs]
                        di = sl["dst"][r, cs]
                        sv = plsc.load_gather(x_local, [si])
                        dv = plsc.load_gather(x_local, [di])
                        eo = pl.ds(r * CCOLS + kk * LANE, LANE)
                        e0 = sl["e0"][eo]
                        wgt = 1.0 / e0 if with_count else e0
                        sl["vals"][r, cs] = (dv - sv) * wgt
                    return 0
                lax.fori_loop(0, CROWS, row, 0)

        def issue_scat(jj, sl):
            chunk = w + NW * jj

            @pl.when(chunk < nchunk)
            def _():
                def rloop(r, _):
                    pltpu.async_copy(sl["vals"].at[r],
                                     acc_sh.at[sl["dst"].at[r]],
                                     sl["semv"], add=True)
                    if with_count:
                        pltpu.async_copy(ones.at[r],
                                         cnt_sh.at[sl["dst"].at[r]],
                                         sl["semo"], add=True)
                    return 0
                lax.fori_loop(0, CROWS, rloop, 0)

        def drain_scat(jj, sl):
            chunk = w + NW * jj

            @pl.when((jj >= 0) & (chunk < nchunk))
            def _():
                def rloop(r, _):
                    pltpu.make_async_copy(sl["vals"].at[r],
                                          acc_sh.at[sl["dst"].at[r]],
                                          sl["semv"]).wait()
                    if with_count:
                        pltpu.make_async_copy(ones.at[r],
                                              cnt_sh.at[sl["dst"].at[r]],
                                              sl["semo"]).wait()
                    return 0
                lax.fori_loop(0, CROWS, rloop, 0)

        start_in(0, slots[0])
        pltpu.sync_copy(x_hbm.at[pl.ds(0, n)], x_local.at[pl.ds(0, n)])
        plsc.subcore_barrier()

        def pair(j2, _):
            a = 2 * j2
            wait_in(a, slots[0])
            compute(a, slots[0])
            drain_scat(a - 1, slots[1])
            start_in(a + 1, slots[1])
            issue_scat(a, slots[0])

            wait_in(a + 1, slots[1])
            compute(a + 1, slots[1])
            drain_scat(a, slots[0])
            start_in(a + 2, slots[0])
            issue_scat(a + 1, slots[1])
            return 0
        lax.fori_loop(0, ch_per_w // 2, pair, 0)
        drain_scat(ch_per_w - 1, slots[1])

        plsc.subcore_barrier()
        pltpu.sync_copy(acc_sh.at[pl.ds(s * sper, sper)],
                        x_local.at[pl.ds(0, sper)])
        pltpu.sync_copy(x_local.at[pl.ds(0, sper)],
                        pa_hbm.at[pl.ds(c * npad + s * sper, sper)])
        if with_count:
            pltpu.sync_copy(cnt_sh.at[pl.ds(s * sper, sper)],
                            x_local.at[pl.ds(0, sper)])
            pltpu.sync_copy(x_local.at[pl.ds(0, sper)],
                            pc_hbm.at[pl.ds(c * npad + s * sper, sper)])

    out_type = (_f32((NC * npad,)), _f32((NC * npad,))) if with_count \
        else _f32((NC * npad,))
    scratch = [pltpu.VMEM_SHARED((npad,), jnp.float32)]
    if with_count:
        scratch.append(pltpu.VMEM_SHARED((npad,), jnp.float32))
    scratch.append(pltpu.VMEM((n,), jnp.float32))
    scratch += [pltpu.VMEM((CROWS, CCOLS), jnp.int32) for _ in range(4)]
    scratch += [pltpu.VMEM((CHUNK,), jnp.float32) for _ in range(2)]
    scratch += [pltpu.VMEM((CROWS, CCOLS), jnp.float32) for _ in range(2)]
    if with_count:
        scratch.append(pltpu.VMEM((CROWS, CCOLS), jnp.float32))
    nsem = 6 if with_count else 4
    scratch += [pltpu.SemaphoreType.DMA for _ in range(nsem)]

    return pl.kernel(
        body, out_type=out_type, mesh=_mesh(), compiler_params=_cparams(),
        scratch_types=scratch)


def _make_w2(n_edges):
    blk = 640000
    grid = n_edges // blk

    def body(e0_ref, e1_ref, o_ref):
        a = e0_ref[...]
        b = e1_ref[...]
        o_ref[...] = (a + b) / (a * b)

    return pl.pallas_call(
        body,
        grid=(grid,),
        in_specs=[pl.BlockSpec((blk,), lambda i: (i,)),
                  pl.BlockSpec((blk,), lambda i: (i,))],
        out_specs=pl.BlockSpec((blk,), lambda i: (i,)),
        out_shape=_f32((n_edges,)),
    )


def _make_glue_t(n, npad):
    per = npad // NW
    groups = per // LANE
    tail = n - per * (NW - 1)

    def body(pa_hbm, pc_hbm, a0_hbm, t_hbm, cnt_hbm,
             b0, b1, c0, c1, ab, tb, cb):
        w = _worker_id()
        sl = pl.ds(w * per, per)
        sl1 = pl.ds(npad + w * per, per)
        pltpu.sync_copy(pa_hbm.at[sl], b0)
        pltpu.sync_copy(pa_hbm.at[sl1], b1)
        pltpu.sync_copy(pc_hbm.at[sl], c0)
        pltpu.sync_copy(pc_hbm.at[sl1], c1)

        @pl.when(w < NW - 1)
        def _():
            pltpu.sync_copy(a0_hbm.at[sl], ab)

        @pl.when(w == NW - 1)
        def _():
            pltpu.sync_copy(a0_hbm.at[pl.ds((NW - 1) * per, tail)],
                            ab.at[pl.ds(0, tail)])

        def grp(g, _):
            d = pl.ds(g * LANE, LANE)
            cc = c0[d] + c1[d]
            t = ab[d] * (b0[d] + b1[d]) / jnp.maximum(cc, 1.0)
            tb[d] = t
            cb[d] = cc
            return 0
        lax.fori_loop(0, groups, grp, 0)
        pltpu.sync_copy(tb, t_hbm.at[sl])
        pltpu.sync_copy(cb, cnt_hbm.at[sl])

    return pl.kernel(
        body,
        out_type=(_f32((npad,)), _f32((npad,))),
        mesh=_mesh(), compiler_params=_cparams(),
        scratch_types=[pltpu.VMEM((per,), jnp.float32) for _ in range(7)],
    )


def _make_final(n, npad):
    per = npad // NW
    groups = per // LANE
    tail = n - per * (NW - 1)

    def body(pb_hbm, cnt_hbm, mask_hbm, fm_hbm, o_hbm,
             b0, b1, cb, mb, fb, ob):
        w = _worker_id()
        sl = pl.ds(w * per, per)
        pltpu.sync_copy(pb_hbm.at[sl], b0)
        pltpu.sync_copy(pb_hbm.at[pl.ds(npad + w * per, per)], b1)
        pltpu.sync_copy(cnt_hbm.at[sl], cb)

        @pl.when(w < NW - 1)
        def _():
            pltpu.sync_copy(mask_hbm.at[sl], mb)
            pltpu.sync_copy(fm_hbm.at[sl], fb)

        @pl.when(w == NW - 1)
        def _():
            tsl = pl.ds((NW - 1) * per, tail)
            pltpu.sync_copy(mask_hbm.at[tsl], mb.at[pl.ds(0, tail)])
            pltpu.sync_copy(fm_hbm.at[tsl], fb.at[pl.ds(0, tail)])

        def grp(g, _):
            d = pl.ds(g * LANE, LANE)
            v = (b0[d] + b1[d]) / jnp.maximum(cb[d], 1.0)
            ob[d] = v * mb[d] - fb[d]
            return 0
        lax.fori_loop(0, groups, grp, 0)
        pltpu.sync_copy(ob, o_hbm.at[sl])

    return pl.kernel(
        body,
        out_type=_f32((npad,)),
        mesh=_mesh(), compiler_params=_cparams(),
        scratch_types=[pltpu.VMEM((per,), jnp.float32) for _ in range(6)],
    )


def _pde_loss(out_x, a_x_x, edge_index, edge_attr, mask, f):
    n = out_x.shape[0]
    n_edges = edge_index.shape[1]
    npad = -(-n // (NW * LANE)) * (NW * LANE)

    x0 = out_x[:, 0]
    a0 = a_x_x[:, 0]
    e0 = edge_attr[:, 0]
    e1 = edge_attr[:, 1]
    fm = mask * jnp.asarray(f, jnp.float32)
    ei4 = edge_index.reshape(2, n_edges // CHUNK, CROWS, CCOLS)

    w2 = _make_w2(n_edges)(e0, e1)
    pa, pc = _make_edge_pass(n, npad, n_edges, True)(x0, ei4, e0)
    t, cnt = _make_glue_t(n, npad)(pa, pc, a0)
    pb = _make_edge_pass(n, npad, n_edges, False)(t, ei4, w2)
    out = _make_final(n, npad)(pb, cnt, mask, fm)
    return out[:n]


def kernel(out_x, a_x_x, edge_index, edge_attr, mask, f):
    return _pde_loss(out_x, a_x_x, edge_index, edge_attr, mask, f)


# speedup vs baseline: 17.0286x; 1.0038x over previous
import jax
import jax.numpy as jnp
from jax import lax
from jax.experimental import pallas as pl
from jax.experimental.pallas import tpu as pltpu
from jax.experimental.pallas import tpu_sc as plsc

NC = 2
NS = 16
NW = NC * NS
LANE = 16
CHUNK = 2048
CROWS = 16
CCOLS = CHUNK // CROWS
GRPS_PER_ROW = CCOLS // LANE


def _mesh():
    return plsc.VectorSubcoreMesh(core_axis_name="c", subcore_axis_name="s")


def _cparams():
    return pltpu.CompilerParams(needs_layout_passes=False)


def _worker_id():
    return lax.axis_index("c") * NS + lax.axis_index("s")


def _f32(shape):
    return jax.ShapeDtypeStruct(shape, jnp.float32)


def _zero_fill(ref, off, n):
    def body(i, _):
        ref[pl.ds(off + i * LANE, LANE)] = jnp.zeros((LANE,), jnp.float32)
        return 0
    lax.fori_loop(0, n // LANE, body, 0)


def _make_edge_pass(n, npad, n_edges, with_count):
    nchunk = n_edges // CHUNK
    ch_per_w = -(-nchunk // NW)
    if ch_per_w % 2:
        ch_per_w += 1
    sper = npad // NS

    def body(x_hbm, ei_hbm, *rest):
        if with_count:
            (e0_hbm, pa_hbm, pc_hbm, acc_sh, cnt_sh, x_local,
             src0, src1, dst0, dst1, e00, e01, vals0, vals1, ones,
             semi0, semi1, semv0, semv1, semo0, semo1) = rest
        else:
            (e0_hbm, pa_hbm, acc_sh, x_local,
             src0, src1, dst0, dst1, e00, e01, vals0, vals1,
             semi0, semi1, semv0, semv1) = rest
            semo0 = semo1 = ones = None
        slots = (
            dict(src=src0, dst=dst0, e0=e00, vals=vals0,
                 semi=semi0, semv=semv0, semo=semo0),
            dict(src=src1, dst=dst1, e0=e01, vals=vals1,
                 semi=semi1, semv=semv1, semo=semo1),
        )
        c = lax.axis_index("c")
        s = lax.axis_index("s")
        w = c * NS + s

        stage = cnt_sh if with_count else acc_sh

        @pl.when(s == 0)
        def _():
            pltpu.sync_copy(x_hbm.at[pl.ds(0, n)], x_local.at[pl.ds(0, n)])
            pltpu.sync_copy(x_local.at[pl.ds(0, n)], stage.at[pl.ds(0, n)])
        plsc.subcore_barrier()

        @pl.when(s != 0)
        def _():
            pltpu.sync_copy(stage.at[pl.ds(0, n)], x_local.at[pl.ds(0, n)])
        plsc.subcore_barrier()

        _zero_fill(e00, 0, CHUNK)

        def zero_shared(sh):
            base = s * sper
            for q in range(sper // CHUNK):
                pltpu.sync_copy(e00.at[pl.ds(0, CHUNK)],
                                sh.at[pl.ds(base + q * CHUNK, CHUNK)])
            rem = sper % CHUNK
            if rem:
                pltpu.sync_copy(
                    e00.at[pl.ds(0, rem)],
                    sh.at[pl.ds(base + (sper // CHUNK) * CHUNK, rem)])

        zero_shared(acc_sh)
        if with_count:
            zero_shared(cnt_sh)

            def ob(i, _):
                r = i // GRPS_PER_ROW
                col = (i % GRPS_PER_ROW) * LANE
                ones[r, pl.ds(col, LANE)] = jnp.ones((LANE,), jnp.float32)
                return 0
            lax.fori_loop(0, CHUNK // LANE, ob, 0)

        def in_pairs(chunk, sl):
            esl = pl.ds(chunk * CHUNK, CHUNK)
            return [
                (ei_hbm.at[0, chunk], sl["src"]),
                (ei_hbm.at[1, chunk], sl["dst"]),
                (e0_hbm.at[esl], sl["e0"]),
            ]

        def start_in(jj, sl):
            chunk = w + NW * jj

            @pl.when(chunk < nchunk)
            def _():
                for a, b in in_pairs(chunk, sl):
                    pltpu.async_copy(a, b, sl["semi"])

        def wait_in(jj, sl):
            chunk = w + NW * jj

            @pl.when(chunk < nchunk)
            def _():
                for a, b in in_pairs(chunk, sl):
                    pltpu.make_async_copy(a, b, sl["semi"]).wait()

        def compute(jj, sl):
            chunk = w + NW * jj

            @pl.when(chunk < nchunk)
            def _():
                def row(r, _):
                    for kk in range(GRPS_PER_ROW):
                        cs = pl.ds(kk * LANE, LANE)
                        si = sl["src"][r, cs]
                        di = sl["dst"][r, cs]
                        sv = plsc.load_gather(x_local, [si])
                        dv = plsc.load_gather(x_local, [di])
                        eo = pl.ds(r * CCOLS + kk * LANE, LANE)
                        e0 = sl["e0"][eo]
                        wgt = 1.0 / e0 if with_count else e0
                        sl["vals"][r, cs] = (dv - sv) * wgt
                    return 0
                lax.fori_loop(0, CROWS, row, 0)

        def issue_scat(jj, sl):
            chunk = w + NW * jj

            @pl.when(chunk < nchunk)
            def _():
                def rloop(r, _):
                    pltpu.async_copy(sl["vals"].at[r],
                                     acc_sh.at[sl["dst"].at[r]],
                                     sl["semv"], add=True)
                    if with_count:
                        pltpu.async_copy(ones.at[r],
                                         cnt_sh.at[sl["dst"].at[r]],
                                         sl["semo"], add=True)
                    return 0
                lax.fori_loop(0, CROWS, rloop, 0)

        def drain_scat(jj, sl):
            chunk = w + NW * jj

            @pl.when((jj >= 0) & (chunk < nchunk))
            def _():
                def rloop(r, _):
                    pltpu.make_async_copy(sl["vals"].at[r],
                                          acc_sh.at[sl["dst"].at[r]],
                                          sl["semv"]).wait()
                    if with_count:
                        pltpu.make_async_copy(ones.at[r],
                                              cnt_sh.at[sl["dst"].at[r]],
                                              sl["semo"]).wait()
                    return 0
                lax.fori_loop(0, CROWS, rloop, 0)

        start_in(0, slots[0])
        plsc.subcore_barrier()

        def pair(j2, _):
            a = 2 * j2
            wait_in(a, slots[0])
            compute(a, slots[0])
            drain_scat(a - 1, slots[1])
            start_in(a + 1, slots[1])
            issue_scat(a, slots[0])

            wait_in(a + 1, slots[1])
            compute(a + 1, slots[1])
            drain_scat(a, slots[0])
            start_in(a + 2, slots[0])
            issue_scat(a + 1, slots[1])
            return 0
        lax.fori_loop(0, ch_per_w // 2, pair, 0)
        drain_scat(ch_per_w - 1, slots[1])

        plsc.subcore_barrier()
        pltpu.sync_copy(acc_sh.at[pl.ds(s * sper, sper)],
                        x_local.at[pl.ds(0, sper)])
        pltpu.sync_copy(x_local.at[pl.ds(0, sper)],
                        pa_hbm.at[pl.ds(c * npad + s * sper, sper)])
        if with_count:
            pltpu.sync_copy(cnt_sh.at[pl.ds(s * sper, sper)],
                            x_local.at[pl.ds(0, sper)])
            pltpu.sync_copy(x_local.at[pl.ds(0, sper)],
                            pc_hbm.at[pl.ds(c * npad + s * sper, sper)])

    out_type = (_f32((NC * npad,)), _f32((NC * npad,))) if with_count \
        else _f32((NC * npad,))
    scratch = [pltpu.VMEM_SHARED((npad,), jnp.float32)]
    if with_count:
        scratch.append(pltpu.VMEM_SHARED((npad,), jnp.float32))
    scratch.append(pltpu.VMEM((n,), jnp.float32))
    scratch += [pltpu.VMEM((CROWS, CCOLS), jnp.int32) for _ in range(4)]
    scratch += [pltpu.VMEM((CHUNK,), jnp.float32) for _ in range(2)]
    scratch += [pltpu.VMEM((CROWS, CCOLS), jnp.float32) for _ in range(2)]
    if with_count:
        scratch.append(pltpu.VMEM((CROWS, CCOLS), jnp.float32))
    nsem = 6 if with_count else 4
    scratch += [pltpu.SemaphoreType.DMA for _ in range(nsem)]

    return pl.kernel(
        body, out_type=out_type, mesh=_mesh(), compiler_params=_cparams(),
        scratch_types=scratch)


def _make_w2(n_edges):
    blk = 640000
    grid = n_edges // blk

    def body(e0_ref, e1_ref, o_ref):
        a = e0_ref[...]
        b = e1_ref[...]
        o_ref[...] = (a + b) / (a * b)

    return pl.pallas_call(
        body,
        grid=(grid,),
        in_specs=[pl.BlockSpec((blk,), lambda i: (i,)),
                  pl.BlockSpec((blk,), lambda i: (i,))],
        out_specs=pl.BlockSpec((blk,), lambda i: (i,)),
        out_shape=_f32((n_edges,)),
    )


def _make_glue_t(n, npad):
    per = npad // NW
    groups = per // LANE
    tail = n - per * (NW - 1)

    def body(pa_hbm, pc_hbm, a0_hbm, t_hbm, cnt_hbm,
             b0, b1, c0, c1, ab, tb, cb):
        w = _worker_id()
        sl = pl.ds(w * per, per)
        sl1 = pl.ds(npad + w * per, per)
        pltpu.sync_copy(pa_hbm.at[sl], b0)
        pltpu.sync_copy(pa_hbm.at[sl1], b1)
        pltpu.sync_copy(pc_hbm.at[sl], c0)
        pltpu.sync_copy(pc_hbm.at[sl1], c1)

        @pl.when(w < NW - 1)
        def _():
            pltpu.sync_copy(a0_hbm.at[sl], ab)

        @pl.when(w == NW - 1)
        def _():
            pltpu.sync_copy(a0_hbm.at[pl.ds((NW - 1) * per, tail)],
                            ab.at[pl.ds(0, tail)])

        def grp(g, _):
            d = pl.ds(g * LANE, LANE)
            cc = c0[d] + c1[d]
            t = ab[d] * (b0[d] + b1[d]) / jnp.maximum(cc, 1.0)
            tb[d] = t
            cb[d] = cc
            return 0
        lax.fori_loop(0, groups, grp, 0)
        pltpu.sync_copy(tb, t_hbm.at[sl])
        pltpu.sync_copy(cb, cnt_hbm.at[sl])

    return pl.kernel(
        body,
        out_type=(_f32((npad,)), _f32((npad,))),
        mesh=_mesh(), compiler_params=_cparams(),
        scratch_types=[pltpu.VMEM((per,), jnp.float32) for _ in range(7)],
    )


def _make_final(n, npad):
    per = npad // NW
    groups = per // LANE
    tail = n - per * (NW - 1)

    def body(pb_hbm, cnt_hbm, mask_hbm, fm_hbm, o_hbm,
             b0, b1, cb, mb, fb, ob):
        w = _worker_id()
        sl = pl.ds(w * per, per)
        pltpu.sync_copy(pb_hbm.at[sl], b0)
        pltpu.sync_copy(pb_hbm.at[pl.ds(npad + w * per, per)], b1)
        pltpu.sync_copy(cnt_hbm.at[sl], cb)

        @pl.when(w < NW - 1)
        def _():
            pltpu.sync_copy(mask_hbm.at[sl], mb)
            pltpu.sync_copy(fm_hbm.at[sl], fb)

        @pl.when(w == NW - 1)
        def _():
            tsl = pl.ds((NW - 1) * per, tail)
            pltpu.sync_copy(mask_hbm.at[tsl], mb.at[pl.ds(0, tail)])
            pltpu.sync_copy(fm_hbm.at[tsl], fb.at[pl.ds(0, tail)])

        def grp(g, _):
            d = pl.ds(g * LANE, LANE)
            v = (b0[d] + b1[d]) / jnp.maximum(cb[d], 1.0)
            ob[d] = v * mb[d] - fb[d]
            return 0
        lax.fori_loop(0, groups, grp, 0)
        pltpu.sync_copy(ob, o_hbm.at[sl])

    return pl.kernel(
        body,
        out_type=_f32((npad,)),
        mesh=_mesh(), compiler_params=_cparams(),
        scratch_types=[pltpu.VMEM((per,), jnp.float32) for _ in range(6)],
    )


def _pde_loss(out_x, a_x_x, edge_index, edge_attr, mask, f):
    n = out_x.shape[0]
    n_edges = edge_index.shape[1]
    npad = -(-n // (NW * LANE)) * (NW * LANE)

    x0 = out_x[:, 0]
    a0 = a_x_x[:, 0]
    e0 = edge_attr[:, 0]
    e1 = edge_attr[:, 1]
    fm = mask * jnp.asarray(f, jnp.float32)
    ei4 = edge_index.reshape(2, n_edges // CHUNK, CROWS, CCOLS)

    w2 = _make_w2(n_edges)(e0, e1)
    pa, pc = _make_edge_pass(n, npad, n_edges, True)(x0, ei4, e0)
    t, cnt = _make_glue_t(n, npad)(pa, pc, a0)
    pb = _make_edge_pass(n, npad, n_edges, False)(t, ei4, w2)
    out = _make_final(n, npad)(pb, cnt, mask, fm)
    return out[:n]


def kernel(out_x, a_x_x, edge_index, edge_attr, mask, f):
    return _pde_loss(out_x, a_x_x, edge_index, edge_attr, mask, f)
